# baseline (speedup 1.0000x reference)
import functools

import jax
import jax.numpy as jnp
from jax import lax
from jax.experimental import pallas as pl
from jax.experimental.pallas import tpu as pltpu
from jax.experimental.pallas import tpu_sc as plsc

_CHUNK = 256
_NBUF = 4


@functools.cache
def _make_gather(V2, D, B, NC, NS):
    NW = NC * NS
    b_per_w = B // NW
    C = _CHUNK
    nbuf = _NBUF
    nchunks = b_per_w // C
    ngroups = nchunks // nbuf
    assert nchunks % nbuf == 0
    mesh = plsc.VectorSubcoreMesh(core_axis_name="c", subcore_axis_name="s")

    @functools.partial(
        pl.kernel,
        mesh=mesh,
        compiler_params=pltpu.CompilerParams(use_tc_tiling_on_sc=False),
        out_type=jax.ShapeDtypeStruct((B, 2, D), jnp.float32),
        scratch_types=(
            [pltpu.VMEM((b_per_w,), jnp.int32)]
            + [pltpu.VMEM((C, 1, D), jnp.float32) for _ in range(nbuf)]
            + [pltpu.SemaphoreType.DMA for _ in range(2 * nbuf)]
        ),
    )
    def k(idx_hbm, table_hbm, out_hbm, idx_v, *bufs_and_sems):
        bufs = bufs_and_sems[:nbuf]
        gsem = bufs_and_sems[nbuf : 2 * nbuf]
        wsem = bufs_and_sems[2 * nbuf :]
        wid = lax.axis_index("s") * NC + lax.axis_index("c")
        base0 = wid * b_per_w

        pltpu.sync_copy(idx_hbm.at[pl.ds(base0, b_per_w)], idx_v)

        def gather_start(i, b):
            idx_slice = idx_v.at[pl.ds(i * C, C)]
            pltpu.async_copy(table_hbm.at[idx_slice], bufs[b], gsem[b])

        def gather_wait(i, b):
            idx_slice = idx_v.at[pl.ds(i * C, C)]
            pltpu.make_async_copy(table_hbm.at[idx_slice], bufs[b], gsem[b]).wait()

        def wb_start(i, b):
            pltpu.async_copy(
                bufs[b],
                out_hbm.at[pl.ds(base0 + i * C, C), pl.ds(0, 1)],
                wsem[b],
            )

        def wb_wait(i, b):
            pltpu.make_async_copy(
                bufs[b],
                out_hbm.at[pl.ds(base0 + i * C, C), pl.ds(0, 1)],
                wsem[b],
            ).wait()

        for b in range(nbuf):
            gather_start(b, b)

        def body(g, carry):
            for b in range(nbuf):
                i = g * nbuf + b
                gather_wait(i, b)
                wb_start(i, b)
                wb_wait(i, b)
                gather_start(i + nbuf, b)
            return carry

        lax.fori_loop(0, ngroups - 1, body, 0)

        for b in range(nbuf):
            i = (ngroups - 1) * nbuf + b
            gather_wait(i, b)
            wb_start(i, b)
        for b in range(nbuf):
            i = (ngroups - 1) * nbuf + b
            wb_wait(i, b)

    return k


def kernel(input, weight):
    V, D = weight.shape
    idx = (input.reshape(-1) * 2).astype(jnp.int32)
    B = idx.shape[0]
    wpad = jnp.pad(weight, ((0, 0), (0, D))).reshape(2 * V, 1, D)
    info = plsc.get_sparse_core_info()
    out = _make_gather(2 * V, D, B, info.num_cores, info.num_subcores)(idx, wpad)
    return out.reshape(input.shape + (2 * D,))[:, :, :D]


# speedup vs baseline: 10.6281x; 10.6281x over previous
import functools

import jax
import jax.numpy as jnp
from jax import lax
from jax.experimental import pallas as pl
from jax.experimental.pallas import tpu as pltpu
from jax.experimental.pallas import tpu_sc as plsc

_CHUNK = 160
_NBUF = 4


@functools.cache
def _make_gather(V, D, B, NC, NS):
    NW = NC * NS
    b_per_w = B // NW
    C = _CHUNK
    nbuf = _NBUF
    nchunks = b_per_w // C
    ngroups = nchunks // nbuf
    assert nchunks % nbuf == 0
    mesh = plsc.VectorSubcoreMesh(core_axis_name="c", subcore_axis_name="s")

    @functools.partial(
        pl.kernel,
        mesh=mesh,
        compiler_params=pltpu.CompilerParams(use_tc_tiling_on_sc=False),
        out_type=jax.ShapeDtypeStruct((B, D), jnp.float32),
        scratch_types=(
            [pltpu.VMEM((b_per_w,), jnp.int32)]
            + [pltpu.VMEM((C, D), jnp.float32) for _ in range(nbuf)]
            + [pltpu.SemaphoreType.DMA for _ in range(2 * nbuf)]
        ),
    )
    def k(idx_hbm, table_hbm, out_hbm, idx_v, *bufs_and_sems):
        bufs = bufs_and_sems[:nbuf]
        gsem = bufs_and_sems[nbuf : 2 * nbuf]
        wsem = bufs_and_sems[2 * nbuf :]
        wid = lax.axis_index("s") * NC + lax.axis_index("c")
        base0 = wid * b_per_w

        pltpu.sync_copy(idx_hbm.at[pl.ds(base0, b_per_w)], idx_v)

        def gather_start(i, b):
            idx_slice = idx_v.at[pl.ds(i * C, C)]
            pltpu.async_copy(table_hbm.at[idx_slice], bufs[b], gsem[b])

        def gather_wait(i, b):
            idx_slice = idx_v.at[pl.ds(i * C, C)]
            pltpu.make_async_copy(table_hbm.at[idx_slice], bufs[b], gsem[b]).wait()

        def wb_start(i, b):
            pltpu.async_copy(bufs[b], out_hbm.at[pl.ds(base0 + i * C, C)], wsem[b])

        def wb_wait(i, b):
            pltpu.make_async_copy(
                bufs[b], out_hbm.at[pl.ds(base0 + i * C, C)], wsem[b]
            ).wait()

        for b in range(nbuf):
            gather_start(b, b)

        def body(g, carry):
            for b in range(nbuf):
                i = g * nbuf + b
                gather_wait(i, b)
                wb_start(i, b)
                wb_wait(i, b)
                gather_start(i + nbuf, b)
            return carry

        lax.fori_loop(0, ngroups - 1, body, 0)

        for b in range(nbuf):
            i = (ngroups - 1) * nbuf + b
            gather_wait(i, b)
            wb_start(i, b)
        for b in range(nbuf):
            i = (ngroups - 1) * nbuf + b
            wb_wait(i, b)

    return k


def kernel(input, weight):
    V, D = weight.shape
    idx = input.reshape(-1).astype(jnp.int32)
    B = idx.shape[0]
    wpad = jnp.pad(weight, ((0, 0), (0, D)))
    info = plsc.get_sparse_core_info()
    out = _make_gather(V, 2 * D, B, info.num_cores, info.num_subcores)(idx, wpad)
    return out.reshape(input.shape + (2 * D,))[:, :, :D]


# speedup vs baseline: 10.6451x; 1.0016x over previous
import functools

import jax
import jax.numpy as jnp
from jax import lax
from jax.experimental import pallas as pl
from jax.experimental.pallas import tpu as pltpu
from jax.experimental.pallas import tpu_sc as plsc

_CHUNK = 128
_NBUF = 4


@functools.cache
def _make_gather(V, D, B, NC, NS):
    NW = NC * NS
    b_per_w = B // NW
    C = _CHUNK
    nbuf = _NBUF
    nchunks = b_per_w // C
    ngroups = nchunks // nbuf
    assert nchunks % nbuf == 0
    mesh = plsc.VectorSubcoreMesh(core_axis_name="c", subcore_axis_name="s")

    @functools.partial(
        pl.kernel,
        mesh=mesh,
        compiler_params=pltpu.CompilerParams(use_tc_tiling_on_sc=False),
        out_type=jax.ShapeDtypeStruct((B, D), jnp.float32),
        scratch_types=(
            [pltpu.VMEM((b_per_w,), jnp.int32)]
            + [pltpu.VMEM((C, D), jnp.float32) for _ in range(nbuf)]
            + [pltpu.SemaphoreType.DMA for _ in range(2 * nbuf)]
        ),
    )
    def k(idx_hbm, table_hbm, out_hbm, idx_v, *bufs_and_sems):
        bufs = bufs_and_sems[:nbuf]
        gsem = bufs_and_sems[nbuf : 2 * nbuf]
        wsem = bufs_and_sems[2 * nbuf :]
        wid = lax.axis_index("s") * NC + lax.axis_index("c")
        base0 = wid * b_per_w

        pltpu.sync_copy(idx_hbm.at[pl.ds(base0, b_per_w)], idx_v)

        def gather_start(i, b):
            idx_slice = idx_v.at[pl.ds(i * C, C)]
            pltpu.async_copy(table_hbm.at[idx_slice], bufs[b], gsem[b])

        def gather_wait(i, b):
            idx_slice = idx_v.at[pl.ds(i * C, C)]
            pltpu.make_async_copy(table_hbm.at[idx_slice], bufs[b], gsem[b]).wait()

        def wb_start(i, b):
            pltpu.async_copy(bufs[b], out_hbm.at[pl.ds(base0 + i * C, C)], wsem[b])

        def wb_wait(i, b):
            pltpu.make_async_copy(
                bufs[b], out_hbm.at[pl.ds(base0 + i * C, C)], wsem[b]
            ).wait()

        for b in range(nbuf):
            gather_start(b, b)

        def body(g, carry):
            for b in range(nbuf):
                i = g * nbuf + b
                gather_wait(i, b)
                wb_start(i, b)
                wb_wait(i, b)
                gather_start(i + nbuf, b)
            return carry

        lax.fori_loop(0, ngroups - 1, body, 0)

        for b in range(nbuf):
            i = (ngroups - 1) * nbuf + b
            gather_wait(i, b)
            wb_start(i, b)
        for b in range(nbuf):
            i = (ngroups - 1) * nbuf + b
            wb_wait(i, b)

    return k


def kernel(input, weight):
    V, D = weight.shape
    idx = input.reshape(-1).astype(jnp.int32)
    B = idx.shape[0]
    wpad = jnp.pad(weight, ((0, 0), (0, D)))
    info = plsc.get_sparse_core_info()
    out = _make_gather(V, 2 * D, B, info.num_cores, info.num_subcores)(idx, wpad)
    return out.reshape(input.shape + (2 * D,))[:, :, :D]
